# Initial kernel scaffold; baseline (speedup 1.0000x reference)
import functools

import numpy as np
import jax
import jax.numpy as jnp
from jax import lax
from jax.experimental import pallas as pl
from jax.experimental.pallas import tpu as pltpu
from jax.experimental.pallas import tpu_sc as plsc

_L2 = [[0, 1], [2], [3, 4], [5, 6, 7], [8], [9, 10], [11, 12],
       [13, 14, 15, 16], [17], [18]]
_NCLS = 19
_NLIST = 49
_DUMMY = 48
_CAP = 256
_K = 200
_NW = 32
_CHUNK = 2048
_ROWS_W = 16
_GCH = 64


def _build_lut() -> np.ndarray:
    grp = np.zeros(_NCLS, np.int32)
    for g, mem in enumerate(_L2):
        for c in mem:
            grp[c] = g
    lut = np.full((6, 32), _DUMMY, np.int32)
    for l in range(_NCLS):
        lut[0, l] = l
        lut[1, l] = 19 + grp[l]
        mem = _L2[grp[l]]
        for k in range(4):
            if k < len(mem) and mem[k] != l:
                lut[2 + k, l] = 29 + mem[k]
    return lut.reshape(-1)


_LUT_NP = _build_lut()
_SC_MESH = plsc.VectorSubcoreMesh(core_axis_name="c", subcore_axis_name="s")


def _tr_body(x_ref, o_ref):
    o_ref[...] = jnp.transpose(x_ref[...], (0, 2, 1))


_tr_call = pl.pallas_call(
    _tr_body,
    grid=(4, 32),
    in_specs=[pl.BlockSpec((1, 256, 512), lambda b, j: (b, 0, j))],
    out_specs=pl.BlockSpec((1, 512, 256), lambda b, j: (b, j, 0)),
    out_shape=jax.ShapeDtypeStruct((4, 128 * 128, 256), jnp.float32),
)


@functools.partial(
    pl.kernel,
    out_type=(
        jax.ShapeDtypeStruct((_NLIST, _NW, _CAP), jnp.int32),
        jax.ShapeDtypeStruct((_NW, 64), jnp.int32),
    ),
    mesh=_SC_MESH,
    scratch_types=[
        pltpu.VMEM((_ROWS_W * 512,), jnp.int32),
        pltpu.VMEM((_NLIST * _CAP,), jnp.int32),
        pltpu.VMEM((64,), jnp.int32),
        pltpu.VMEM((192,), jnp.int32),
        pltpu.SemaphoreType.DMA,
    ],
)
def _scan_kernel(labels_hbm, lut_hbm, lists_hbm, counts_hbm,
                 rows_v, lists_v, counts_v, lut_v, sem):
    cid = lax.axis_index("c")
    sid = lax.axis_index("s")
    w = sid * 2 + cid
    lanes = lax.iota(jnp.int32, 16)
    zeros16 = jnp.zeros((16,), jnp.int32)

    for t in range(4):
        counts_v[pl.ds(t * 16, 16)] = zeros16
    pltpu.sync_copy(lut_hbm, lut_v)

    cps = []
    for r in range(_ROWS_W):
        R = w * _ROWS_W + r
        src = (R // 128) * 512 + (R % 128) * 4
        cps.append(pltpu.async_copy(labels_hbm.at[src],
                                    rows_v.at[pl.ds(r * 512, 512)], sem))
    for cp in cps:
        cp.wait()

    dc0, _ = plsc.scan_count(zeros16)
    base = jnp.min(dc0)

    def step(i, carry):
        col = (i % 8) * 16
        lab = plsc.load_gather(rows_v, [(i // 8) * 512 + (col + lanes) * 4])
        gvec = w * _CHUNK + i * 16 + lanes
        for p in range(6):
            tgt = plsc.load_gather(lut_v, [p * 32 + lab])
            cnt = plsc.load_gather(counts_v, [tgt])
            dc, lastm = plsc.scan_count(tgt)
            rank = cnt + dc - base
            plsc.store_scatter(lists_v, [tgt * _CAP + rank], gvec,
                               mask=rank < _CAP)
            plsc.store_scatter(counts_v, [tgt], rank + 1, mask=lastm)
        return carry

    lax.fori_loop(0, _CHUNK // 16, step, jnp.int32(0))

    cps = []
    for L in range(_NLIST - 1):
        cps.append(pltpu.async_copy(lists_v.at[pl.ds(L * _CAP, _CAP)],
                                    lists_hbm.at[L, w], sem))
    cps.append(pltpu.async_copy(counts_v, counts_hbm.at[w], sem))
    for cp in cps:
        cp.wait()


@functools.partial(
    pl.kernel,
    out_type=(
        jax.ShapeDtypeStruct((16,), jnp.float32),
        jax.ShapeDtypeStruct((16,), jnp.int32),
    ),
    mesh=_SC_MESH,
    scratch_types=[
        pltpu.VMEM((_NW * 64,), jnp.int32),
        pltpu.VMEM((_NW * _CAP,), jnp.int32),
        pltpu.VMEM((_CAP,), jnp.int32),
        pltpu.VMEM((_CAP,), jnp.int32),
        pltpu.VMEM((_CAP,), jnp.int32),
        pltpu.VMEM((_GCH, 256), jnp.float32),
        pltpu.VMEM((_GCH, 256), jnp.float32),
        pltpu.VMEM((_GCH, 256), jnp.float32),
        pltpu.VMEM((192,), jnp.int32),
        pltpu.VMEM((16,), jnp.int32),
        pltpu.VMEM((32 * 16,), jnp.float32),
        pltpu.VMEM((16,), jnp.float32),
        pltpu.VMEM((16,), jnp.int32),
        pltpu.VMEM_SHARED((32, 16), jnp.float32),
        pltpu.SemaphoreType.DMA,
    ],
)
def _loss_kernel(emb_hbm, lists_hbm, counts_hbm, lut_hbm, mt_hbm,
                 loss_hbm, cnt_hbm,
                 cnts_v, tmpl_v, la_v, lp_v, ln_v, ra_v, rp_v, rn_v,
                 lut_v, mt_v, red_v, stf_v, sti_v, shared_v, sem):
    cid = lax.axis_index("c")
    sid = lax.axis_index("s")
    lanes = lax.iota(jnp.int32, 16)
    flanes = lanes.astype(jnp.float32)

    def bcast(x):
        return jnp.full((16,), 1, jnp.int32) * x

    @pl.when(cid == 0)
    def _work():
        pltpu.sync_copy(counts_hbm, cnts_v)
        pltpu.sync_copy(lut_hbm, lut_v)
        pltpu.sync_copy(mt_hbm, mt_v)
        mt = jnp.min(mt_v[...])

        def total_count(lid):
            t1 = jnp.sum(plsc.load_gather(cnts_v, [lanes * 64 + lid]))
            t2 = jnp.sum(plsc.load_gather(cnts_v, [(lanes + 16) * 64 + lid]))
            return t1 + t2

        def merge(lid, dst):
            for v in range(_CAP // 16):
                dst[pl.ds(v * 16, 16)] = jnp.zeros((16,), jnp.int32)
            cps = []
            for s in range(_NW):
                cps.append(pltpu.async_copy(
                    lists_hbm.at[lid, s], tmpl_v.at[pl.ds(s * _CAP, _CAP)],
                    sem))
            for cp in cps:
                cp.wait()

            def body(s, prefix):
                cnt_s = jnp.min(plsc.load_gather(cnts_v, [bcast(s * 64 + lid)]))
                cap_s = jnp.minimum(cnt_s, _CAP)
                for v in range(13):
                    lp = v * 16 + lanes
                    src = plsc.load_gather(tmpl_v, [s * _CAP + lp])
                    dest = prefix + lp
                    plsc.store_scatter(dst, [dest], src,
                                       mask=(lp < cap_s) & (dest < _K))
                return prefix + cap_s

            lax.fori_loop(0, _NW, body, jnp.int32(0))

        def class_loss(c):
            gid = jnp.min(plsc.load_gather(lut_v, [bcast(32 + c)]))
            merge(c, la_v)
            merge(29 + c, lp_v)
            merge(gid, ln_v)
            ta = total_count(c)
            tp = total_count(29 + c)
            tn = total_count(gid)
            msize = jnp.minimum(jnp.minimum(ta, tp), jnp.minimum(tn, mt))
            vcount = jnp.minimum(msize, _K)
            clsum = jnp.float32(0.0)
            for ch in range(_CAP // _GCH):
                nrows = jnp.clip(vcount - ch * _GCH, 0, _GCH)
                cpa = pltpu.async_copy(
                    emb_hbm.at[la_v.at[pl.ds(ch * _GCH, _GCH)]], ra_v, sem)
                cpp = pltpu.async_copy(
                    emb_hbm.at[lp_v.at[pl.ds(ch * _GCH, _GCH)]], rp_v, sem)
                cpn = pltpu.async_copy(
                    emb_hbm.at[ln_v.at[pl.ds(ch * _GCH, _GCH)]], rn_v, sem)
                cpa.wait()
                cpp.wait()
                cpn.wait()

                def dot_body(j, acc):
                    jb = bcast(j)
                    accp = jnp.zeros((16,), jnp.float32)
                    accn = jnp.zeros((16,), jnp.float32)
                    for k in range(16):
                        col = k * 16 + lanes
                        a = plsc.load_gather(ra_v, [jb, col])
                        p = plsc.load_gather(rp_v, [jb, col])
                        n = plsc.load_gather(rn_v, [jb, col])
                        accp = accp + a * p
                        accn = accn + a * n
                    tl = jnp.maximum(jnp.sum(accn - accp) + jnp.float32(0.6),
                                     jnp.float32(0.0))
                    return acc + tl

                clsum = lax.fori_loop(0, nrows, dot_body, clsum)
            den = jnp.maximum(msize, 1).astype(jnp.float32)
            lossc = jnp.where(msize > 0, clsum / den, jnp.float32(0.0))
            hasc = jnp.where(msize > 0, jnp.float32(1.0), jnp.float32(0.0))
            vec = jnp.where(lanes == 0, lossc,
                            jnp.where(lanes == 1, hasc, jnp.float32(0.0)))
            stf_v[...] = vec
            pltpu.sync_copy(stf_v, shared_v.at[c])

        class_loss(sid)

        @pl.when(sid < 3)
        def _second():
            class_loss(sid + 16)

        @pl.when(sid == 0)
        def _zero_tail():
            stf_v[...] = jnp.zeros((16,), jnp.float32)
            for r in range(_NCLS, 32):
                pltpu.sync_copy(stf_v, shared_v.at[r])

    plsc.subcore_barrier()

    @pl.when((cid == 0) & (sid == 0))
    def _reduce():
        pltpu.sync_copy(shared_v, red_v)
        l1 = jnp.sum(plsc.load_gather(red_v, [lanes * 16]))
        l2 = jnp.sum(plsc.load_gather(red_v, [(lanes + 16) * 16]))
        h1 = jnp.sum(plsc.load_gather(red_v, [lanes * 16 + 1]))
        h2 = jnp.sum(plsc.load_gather(red_v, [(lanes + 16) * 16 + 1]))
        total = l1 + l2
        hsum = h1 + h2
        loss = total / jnp.maximum(hsum, jnp.float32(1.0))
        stf_v[...] = jnp.zeros((16,), jnp.float32) + loss
        pltpu.sync_copy(stf_v, loss_hbm)
        sti_v[...] = jnp.zeros((16,), jnp.int32) + hsum.astype(jnp.int32)
        pltpu.sync_copy(sti_v, cnt_hbm)


def kernel(embedding, labels, max_triplet):
    emb3 = embedding.reshape(4, 256, 128 * 128)
    emb_t = _tr_call(emb3).reshape(4 * 128 * 128, 256)
    lab2 = labels.reshape(4 * 512, 512)
    lut = jnp.asarray(_LUT_NP)
    lists, counts = _scan_kernel(lab2, lut)
    mt16 = jnp.full((16,), max_triplet, jnp.int32)
    loss16, cnt16 = _loss_kernel(emb_t, lists, counts.reshape(-1), lut, mt16)
    return loss16[0], cnt16[0:1]


# speedup vs baseline: 6.1319x; 6.1319x over previous
import functools

import numpy as np
import jax
import jax.numpy as jnp
from jax import lax
from jax.experimental import pallas as pl
from jax.experimental.pallas import tpu as pltpu
from jax.experimental.pallas import tpu_sc as plsc

_L2 = [[0, 1], [2], [3, 4], [5, 6, 7], [8], [9, 10], [11, 12],
       [13, 14, 15, 16], [17], [18]]
_NCLS = 19
_NLIST = 49
_DUMMY = 48
_CAP = 256
_K = 200
_NW = 32
_CHUNK = 2048
_ROWS_W = 16
_GCH = 64


def _build_lut() -> np.ndarray:
    grp = np.zeros(_NCLS, np.int32)
    for g, mem in enumerate(_L2):
        for c in mem:
            grp[c] = g
    lut = np.full((6, 32), _DUMMY, np.int32)
    for l in range(_NCLS):
        lut[0, l] = l
        lut[1, l] = 19 + grp[l]
        mem = _L2[grp[l]]
        for k in range(4):
            if k < len(mem) and mem[k] != l:
                lut[2 + k, l] = 29 + mem[k]
    return lut.reshape(-1)


_LUT_NP = _build_lut()


def _tr_body(x_ref, o_ref):
    o_ref[...] = jnp.transpose(x_ref[...], (0, 2, 1))


_tr_call = pl.pallas_call(
    _tr_body,
    grid=(4, 32),
    in_specs=[pl.BlockSpec((1, 256, 512), lambda b, j: (b, 0, j))],
    out_specs=pl.BlockSpec((1, 512, 256), lambda b, j: (b, j, 0)),
    out_shape=jax.ShapeDtypeStruct((4, 128 * 128, 256), jnp.float32),
)


@functools.lru_cache(maxsize=1)
def _build_sc_kernels():
    mesh = plsc.VectorSubcoreMesh(core_axis_name="c", subcore_axis_name="s")
    scan = _make_scan_kernel(mesh)
    loss = _make_loss_kernel(mesh)
    red = _make_reduce_kernel(mesh)
    return scan, loss, red


def _make_scan_kernel(mesh):
    return functools.partial(
        pl.kernel,
        out_type=(
            jax.ShapeDtypeStruct((_NLIST, _NW, _CAP), jnp.int32),
            jax.ShapeDtypeStruct((_NW, 64), jnp.int32),
        ),
        mesh=mesh,
        scratch_types=[
        pltpu.VMEM((_ROWS_W * 512,), jnp.int32),
        pltpu.VMEM((_NLIST * _CAP,), jnp.int32),
        pltpu.VMEM((64,), jnp.int32),
        pltpu.VMEM((192,), jnp.int32),
            pltpu.SemaphoreType.DMA,
        ],
        compiler_params=pltpu.CompilerParams(needs_layout_passes=False),
    )(_scan_body)


def _scan_body(labels_hbm, lut_hbm, lists_hbm, counts_hbm,
               rows_v, lists_v, counts_v, lut_v, sem):
    cid = lax.axis_index("c")
    sid = lax.axis_index("s")
    w = sid * 2 + cid
    lanes = lax.iota(jnp.int32, 16)
    zeros16 = jnp.zeros((16,), jnp.int32)

    for t in range(4):
        counts_v[pl.ds(t * 16, 16)] = zeros16
    pltpu.sync_copy(lut_hbm, lut_v)

    cps = []
    for r in range(_ROWS_W):
        R = w * _ROWS_W + r
        src = (R // 128) * 512 + (R % 128) * 4
        cps.append(pltpu.async_copy(labels_hbm.at[src],
                                    rows_v.at[pl.ds(r * 512, 512)], sem))
    for cp in cps:
        cp.wait()

    dc0, _ = plsc.scan_count(zeros16)
    base = jnp.min(dc0)

    def step(i, carry):
        col = (i % 8) * 16
        lab = plsc.load_gather(rows_v, [(i // 8) * 512 + (col + lanes) * 4])
        gvec = w * _CHUNK + i * 16 + lanes
        for p in range(6):
            tgt = plsc.load_gather(lut_v, [p * 32 + lab])
            cnt = plsc.load_gather(counts_v, [tgt])
            dc, lastm = plsc.scan_count(tgt)
            rank = cnt + dc - base
            plsc.store_scatter(lists_v, [tgt * _CAP + rank], gvec,
                               mask=rank < _CAP)
            plsc.store_scatter(counts_v, [tgt], rank + 1, mask=lastm)
        return carry

    lax.fori_loop(0, _CHUNK // 16, step, jnp.int32(0))

    cps = []
    for L in range(_NLIST - 1):
        cps.append(pltpu.async_copy(lists_v.at[pl.ds(L * _CAP, _CAP)],
                                    lists_hbm.at[L, w], sem))
    cps.append(pltpu.async_copy(counts_v, counts_hbm.at[w], sem))
    for cp in cps:
        cp.wait()


def _make_loss_kernel(mesh):
    return functools.partial(
        pl.kernel,
        out_type=(
            jax.ShapeDtypeStruct((_NCLS, _CAP), jnp.float32),
            jax.ShapeDtypeStruct((16, _CAP), jnp.int32),
        ),
        mesh=mesh,
        scratch_types=[
        pltpu.VMEM((_NW * 64,), jnp.int32),
        pltpu.VMEM((_NW * _CAP,), jnp.int32),
        pltpu.VMEM((_CAP,), jnp.int32),
        pltpu.VMEM((_CAP,), jnp.int32),
        pltpu.VMEM((_CAP,), jnp.int32),
        pltpu.VMEM((_CAP,), jnp.int32),
        pltpu.VMEM((_GCH, 256), jnp.float32),
        pltpu.VMEM((_GCH, 256), jnp.float32),
        pltpu.VMEM((_GCH, 256), jnp.float32),
        pltpu.VMEM((192,), jnp.int32),
        pltpu.VMEM((16,), jnp.int32),
        pltpu.VMEM((_GCH,), jnp.float32),
        pltpu.VMEM((16,), jnp.float32),
            pltpu.SemaphoreType.DMA,
        ],
        compiler_params=pltpu.CompilerParams(needs_layout_passes=False),
    )(_loss_body)


def _loss_body(emb_hbm, lists_hbm, counts_hbm, lut_hbm, mt_hbm,
               tlrow_hbm, bounce_hbm,
               cnts_v, tmpl_v, la_v, lp_v, ln_v, mg_v, ra_v, rp_v, rn_v,
               lut_v, mt_v, tl_v, stf_v, sem):
    cid = lax.axis_index("c")
    sid = lax.axis_index("s")
    lanes = lax.iota(jnp.int32, 16)

    def bcast(x):
        return jnp.full((16,), 1, jnp.int32) * x

    @pl.when(cid == 0)
    def _work():
        pltpu.sync_copy(counts_hbm, cnts_v)
        pltpu.sync_copy(lut_hbm, lut_v)

        def merge(lid, dst):
            for v in range(_CAP // 16):
                mg_v[pl.ds(v * 16, 16)] = jnp.zeros((16,), jnp.int32)
            cps = []
            for s in range(_NW):
                cps.append(pltpu.async_copy(
                    lists_hbm.at[lid, s], tmpl_v.at[pl.ds(s * _CAP, _CAP)],
                    sem))
            for cp in cps:
                cp.wait()

            def body(s, prefix):
                cnt_s = jnp.min(plsc.load_gather(cnts_v, [bcast(s * 64 + lid)]))
                cap_s = jnp.minimum(cnt_s, _CAP)
                for v in range(13):
                    lp = v * 16 + lanes
                    src = plsc.load_gather(tmpl_v, [s * _CAP + lp])
                    dest = prefix + lp
                    plsc.store_scatter(mg_v, [dest], src,
                                       mask=(lp < cap_s) & (dest < _K))
                return prefix + cap_s

            lax.fori_loop(0, _NW, body, jnp.int32(0))
            pltpu.sync_copy(mg_v, bounce_hbm.at[sid])
            pltpu.sync_copy(bounce_hbm.at[sid], dst)

        def class_loss(c):
            gid = jnp.min(plsc.load_gather(lut_v, [bcast(32 + c)]))
            merge(c, la_v)
            merge(29 + c, lp_v)
            merge(gid, ln_v)
            for ch in range(_CAP // _GCH):
                pltpu.async_copy(
                    emb_hbm.at[la_v.at[pl.ds(ch * _GCH, _GCH)]], ra_v,
                    sem).wait()
                pltpu.async_copy(
                    emb_hbm.at[lp_v.at[pl.ds(ch * _GCH, _GCH)]], rp_v,
                    sem).wait()
                pltpu.async_copy(
                    emb_hbm.at[ln_v.at[pl.ds(ch * _GCH, _GCH)]], rn_v,
                    sem).wait()

                def dot_body(j, carry):
                    jb = bcast(j)
                    accp = jnp.zeros((16,), jnp.float32)
                    accn = jnp.zeros((16,), jnp.float32)
                    for k in range(16):
                        col = k * 16 + lanes
                        a = plsc.load_gather(ra_v, [jb, col])
                        p = plsc.load_gather(rp_v, [jb, col])
                        n = plsc.load_gather(rn_v, [jb, col])
                        accp = accp + a * p
                        accn = accn + a * n
                    tl = jnp.maximum(jnp.sum(accn - accp) + jnp.float32(0.6),
                                     jnp.float32(0.0))
                    plsc.store_scatter(tl_v, [bcast(j)],
                                       jnp.zeros((16,), jnp.float32) + tl,
                                       mask=lanes == 0)
                    return carry

                lax.fori_loop(0, _GCH, dot_body, jnp.int32(0))
                pltpu.sync_copy(tl_v, tlrow_hbm.at[c, pl.ds(ch * _GCH, _GCH)])

        class_loss(sid)

        @pl.when(sid < 3)
        def _second():
            class_loss(sid + 16)


def _make_reduce_kernel(mesh):
    return functools.partial(
        pl.kernel,
        out_type=(
            jax.ShapeDtypeStruct((16,), jnp.float32),
            jax.ShapeDtypeStruct((16,), jnp.int32),
        ),
        mesh=mesh,
        scratch_types=[
            pltpu.VMEM((_NW * 64,), jnp.int32),
            pltpu.VMEM((_CAP,), jnp.float32),
            pltpu.VMEM((192,), jnp.int32),
            pltpu.VMEM((16,), jnp.int32),
            pltpu.VMEM((16,), jnp.float32),
            pltpu.VMEM((16,), jnp.int32),
            pltpu.SemaphoreType.DMA,
        ],
        compiler_params=pltpu.CompilerParams(needs_layout_passes=False),
    )(_reduce_body)


def _reduce_body(counts_hbm, lut_hbm, mt_hbm, tlrow_hbm, loss_hbm, cnt_hbm,
                 cnts_v, tlr_v, lut_v, mt_v, stf_v, sti_v, sem):
    cid = lax.axis_index("c")
    sid = lax.axis_index("s")
    lanes = lax.iota(jnp.int32, 16)
    flanes = lanes.astype(jnp.float32)

    @pl.when((cid == 0) & (sid == 0))
    def _reduce():
        pltpu.sync_copy(counts_hbm, cnts_v)
        pltpu.sync_copy(lut_hbm, lut_v)
        pltpu.sync_copy(mt_hbm, mt_v)
        mt = jnp.min(mt_v[...])
        zf = jnp.zeros((16,), jnp.float32)

        def bcast(x):
            return jnp.full((16,), 1, jnp.int32) * x

        def total_count(lid):
            t1 = jnp.sum(plsc.load_gather(cnts_v, [lanes * 64 + lid]))
            t2 = jnp.sum(plsc.load_gather(cnts_v, [(lanes + 16) * 64 + lid]))
            return t1 + t2

        lsum = zf
        hsum = zf
        for c in range(_NCLS):
            pltpu.sync_copy(tlrow_hbm.at[pl.ds(c * _CAP, _CAP)], tlr_v)
            gid = jnp.min(plsc.load_gather(lut_v, [bcast(32 + c)]))
            ta = total_count(c)
            tp = total_count(29 + c)
            tn = total_count(gid)
            ms = jnp.minimum(jnp.minimum(ta, tp), jnp.minimum(tn, mt))
            vc = jnp.minimum(ms, _K).astype(jnp.float32)
            s = jnp.float32(0.0)
            for v in range(_CAP // 16):
                pos = jnp.float32(v * 16) + flanes
                tv = tlr_v[pl.ds(v * 16, 16)]
                s = s + jnp.sum(jnp.where(pos < vc, tv, zf))
            lv = (zf + s) / jnp.maximum(zf + ms.astype(jnp.float32), 1.0)
            has = ms > 0
            lsum = lsum + jnp.where(has, lv, zf)
            hsum = hsum + jnp.where(has, zf + 1.0, zf)
        stf_v[...] = lsum / jnp.maximum(hsum, 1.0)
        pltpu.sync_copy(stf_v, loss_hbm)
        sti_v[...] = hsum.astype(jnp.int32)
        pltpu.sync_copy(sti_v, cnt_hbm)


def kernel(embedding, labels, max_triplet):
    emb3 = embedding.reshape(4, 256, 128 * 128)
    emb_t = _tr_call(emb3).reshape(4 * 128 * 128, 256)
    lab2 = labels.reshape(4 * 512, 512)
    lut = jnp.asarray(_LUT_NP)
    scan_k, loss_k, red_k = _build_sc_kernels()
    lists, counts = scan_k(lab2, lut)
    mt16 = jnp.full((16,), max_triplet, jnp.int32)
    tlrows, _ = loss_k(emb_t, lists, counts.reshape(-1), lut, mt16)
    loss16, cnt16 = red_k(counts.reshape(-1), lut, mt16, tlrows.reshape(-1))
    return loss16[0], cnt16[0:1]
